# Initial kernel scaffold; baseline (speedup 1.0000x reference)
import jax
import jax.numpy as jnp
from jax.experimental import pallas as pl


def kernel(hidden_states, style_emb, Wg, W1, b1, W2, b2):
    raise NotImplementedError("write your pallas kernel here")


# speedup vs baseline: 1.8390x; 1.8390x over previous
import functools

import jax
import jax.numpy as jnp
from jax import lax
from jax.experimental import pallas as pl
from jax.experimental.pallas import tpu as pltpu
from jax.experimental.pallas import tpu_sc as plsc

BB = 2
LL = 2048
DD = 768
II = 3072
EE = 8
NTOK = BB * LL
TILE = 256
NTILES = (2 * NTOK) // TILE + EE
PROWS = NTILES * TILE

NC = 2
NS = 16
NW = NC * NS
TW = NTOK // NW
CH = 64


def _cumsum_rows(a):
    n = a.shape[0]
    s = 1
    while s < n:
        a = a + jnp.concatenate([jnp.zeros((s, a.shape[1]), a.dtype), a[:-s, :]], axis=0)
        s *= 2
    return a


def _cumsum_lanes(a):
    s = 1
    while s < a.shape[1]:
        a = a + jnp.concatenate([jnp.zeros((1, s), a.dtype), a[:, :-s]], axis=1)
        s *= 2
    return a


def _router_body(x_ref, style_ref, wg_ref, pos0_ref, pos1_ref, w0_ref, w1_ref,
                 te_ref, usage_ref):
    neg_inf = jnp.float32(-jnp.inf)
    g = jnp.dot(x_ref[...], wg_ref[...], preferred_element_type=jnp.float32)
    lane = lax.broadcasted_iota(jnp.int32, (NTOK, EE), 1)
    v0 = jnp.max(g, axis=1, keepdims=True)
    e0 = jnp.min(jnp.where(g == v0, lane, EE), axis=1, keepdims=True)
    m = jnp.where(lane == e0, neg_inf, g)
    v1 = jnp.max(m, axis=1, keepdims=True)
    e1 = jnp.min(jnp.where(m == v1, lane, EE), axis=1, keepdims=True)
    ew = jnp.exp(v1 - v0)
    w0 = 1.0 / (1.0 + ew)
    w1 = ew / (1.0 + ew)

    oh0 = (lane == e0).astype(jnp.int32)
    oh1 = (lane == e1).astype(jnp.int32)
    c0 = _cumsum_rows(oh0)
    c1 = _cumsum_rows(oh1)
    tot0 = c0[NTOK - 1:NTOK, :]
    tot1 = c1[NTOK - 1:NTOK, :]
    counts = tot0 + tot1
    rank0 = jnp.sum(oh0 * (c0 - 1), axis=1, keepdims=True)
    rank1 = jnp.sum(oh1 * (tot0 + c1 - 1), axis=1, keepdims=True)

    pc = ((counts + (TILE - 1)) // TILE) * TILE
    off = _cumsum_lanes(pc) - pc
    pos0_ref[...] = jnp.sum(oh0 * off, axis=1, keepdims=True) + rank0
    pos1_ref[...] = jnp.sum(oh1 * off, axis=1, keepdims=True) + rank1
    w0_ref[...] = jnp.broadcast_to(w0, (NTOK, 128))
    w1_ref[...] = jnp.broadcast_to(w1, (NTOK, 128))

    endpos = off + pc
    tstart = lax.broadcasted_iota(jnp.int32, (NTILES, EE), 0) * TILE
    te = jnp.sum((tstart >= endpos).astype(jnp.int32), axis=1, keepdims=True)
    te_ref[...] = jnp.minimum(te, EE - 1)

    sg = jnp.dot(style_ref[...], wg_ref[...], preferred_element_type=jnp.float32)
    slane = lax.broadcasted_iota(jnp.int32, (8, EE), 1)
    sv0 = jnp.max(sg, axis=1, keepdims=True)
    se0 = jnp.min(jnp.where(sg == sv0, slane, EE), axis=1, keepdims=True)
    sm = jnp.where(slane == se0, neg_inf, sg)
    sv1 = jnp.max(sm, axis=1, keepdims=True)
    se1 = jnp.min(jnp.where(sm == sv1, slane, EE), axis=1, keepdims=True)
    rowvalid = lax.broadcasted_iota(jnp.int32, (8, EE), 0) < BB
    soh = ((slane == se0) | (slane == se1)) & rowvalid
    usage_ref[...] = counts.astype(jnp.float32) + LL * jnp.sum(
        soh.astype(jnp.float32), axis=0, keepdims=True)


def _router(x, style8, wg):
    return pl.pallas_call(
        _router_body,
        out_shape=(
            jax.ShapeDtypeStruct((NTOK, 1), jnp.int32),
            jax.ShapeDtypeStruct((NTOK, 1), jnp.int32),
            jax.ShapeDtypeStruct((NTOK, 128), jnp.float32),
            jax.ShapeDtypeStruct((NTOK, 128), jnp.float32),
            jax.ShapeDtypeStruct((NTILES, 1), jnp.int32),
            jax.ShapeDtypeStruct((1, EE), jnp.float32),
        ),
    )(x, style8, wg)


@functools.cache
def _dispatch_kernel():
    return functools.partial(
        pl.kernel,
        out_type=[
            jax.ShapeDtypeStruct((PROWS, DD), jnp.float32),
            jax.ShapeDtypeStruct((PROWS, 128), jnp.float32),
        ],
        mesh=plsc.VectorSubcoreMesh(core_axis_name="c", subcore_axis_name="s"),
        scratch_types=[
            pltpu.VMEM((CH, DD), jnp.float32),
            pltpu.VMEM((CH,), jnp.int32),
            pltpu.VMEM((CH,), jnp.int32),
            pltpu.VMEM((CH, 128), jnp.float32),
            pltpu.VMEM((CH, 128), jnp.float32),
            pltpu.SemaphoreType.DMA,
            pltpu.SemaphoreType.DMA,
        ],
    )(_dispatch_body)


def _dispatch_body(x_hbm, pos0_hbm, pos1_hbm, w0_hbm, w1_hbm, xs_hbm, ws_hbm,
                   buf, idx0, idx1, wb0, wb1, sem0, sem1):
    wid = lax.axis_index("s") * NC + lax.axis_index("c")
    base = wid * TW
    for c in range(TW // CH):
        cb = base + c * CH
        pltpu.sync_copy(x_hbm.at[pl.ds(cb, CH)], buf)
        pltpu.sync_copy(pos0_hbm.at[pl.ds(cb, CH)], idx0)
        pltpu.sync_copy(pos1_hbm.at[pl.ds(cb, CH)], idx1)
        pltpu.sync_copy(w0_hbm.at[pl.ds(cb, CH)], wb0)
        pltpu.sync_copy(w1_hbm.at[pl.ds(cb, CH)], wb1)
        c0 = pltpu.async_copy(buf, xs_hbm.at[idx0], sem0)
        c1 = pltpu.async_copy(buf, xs_hbm.at[idx1], sem1)
        c2 = pltpu.async_copy(wb0, ws_hbm.at[idx0], sem0)
        c3 = pltpu.async_copy(wb1, ws_hbm.at[idx1], sem1)
        c0.wait()
        c1.wait()
        c2.wait()
        c3.wait()


def _mm_body(te_ref, xs_ref, w1_ref, b1_ref, w2_ref, b2_ref, ws_ref, out_ref):
    h = jnp.dot(xs_ref[...], w1_ref[0], preferred_element_type=jnp.float32)
    h = h + b1_ref[0]
    h = jax.nn.gelu(h, approximate=True)
    y = jnp.dot(h, w2_ref[0], preferred_element_type=jnp.float32)
    y = y + b2_ref[0]
    out_ref[...] = y * ws_ref[:, 0:1]


def _grouped_mm(te, xs, w1, b1, w2, b2, ws):
    grid_spec = pltpu.PrefetchScalarGridSpec(
        num_scalar_prefetch=1,
        grid=(NTILES,),
        in_specs=[
            pl.BlockSpec((TILE, DD), lambda i, te: (i, 0)),
            pl.BlockSpec((1, DD, II), lambda i, te: (te[i], 0, 0)),
            pl.BlockSpec((1, 1, II), lambda i, te: (te[i], 0, 0)),
            pl.BlockSpec((1, II, DD), lambda i, te: (te[i], 0, 0)),
            pl.BlockSpec((1, 1, DD), lambda i, te: (te[i], 0, 0)),
            pl.BlockSpec((TILE, 128), lambda i, te: (i, 0)),
        ],
        out_specs=pl.BlockSpec((TILE, DD), lambda i, te: (i, 0)),
    )
    return pl.pallas_call(
        _mm_body,
        grid_spec=grid_spec,
        out_shape=jax.ShapeDtypeStruct((PROWS, DD), jnp.float32),
    )(te, xs, w1, b1, w2, b2, ws)


@functools.cache
def _combine_kernel():
    return functools.partial(
        pl.kernel,
        out_type=jax.ShapeDtypeStruct((NTOK, DD), jnp.float32),
        mesh=plsc.VectorSubcoreMesh(core_axis_name="c", subcore_axis_name="s"),
        scratch_types=[
            pltpu.VMEM((CH, DD), jnp.float32),
            pltpu.VMEM((CH, DD), jnp.float32),
            pltpu.VMEM((CH,), jnp.int32),
            pltpu.VMEM((CH,), jnp.int32),
            pltpu.SemaphoreType.DMA,
            pltpu.SemaphoreType.DMA,
        ],
    )(_combine_body)


def _combine_body(ys_hbm, pos0_hbm, pos1_hbm, out_hbm,
                  buf0, buf1, idx0, idx1, sem0, sem1):
    wid = lax.axis_index("s") * NC + lax.axis_index("c")
    base = wid * TW
    for c in range(TW // CH):
        cb = base + c * CH
        pltpu.sync_copy(pos0_hbm.at[pl.ds(cb, CH)], idx0)
        pltpu.sync_copy(pos1_hbm.at[pl.ds(cb, CH)], idx1)
        g0 = pltpu.async_copy(ys_hbm.at[idx0], buf0, sem0)
        g1 = pltpu.async_copy(ys_hbm.at[idx1], buf1, sem1)
        g0.wait()
        g1.wait()

        def row_body(i, carry):
            for j in range(DD // 16):
                plsc.addupdate(buf0.at[i, pl.ds(j * 16, 16)],
                               buf1[i, pl.ds(j * 16, 16)])
            return carry

        lax.fori_loop(0, CH, row_body, 0)
        pltpu.sync_copy(buf0, out_hbm.at[pl.ds(cb, CH)])


def kernel(hidden_states, style_emb, Wg, W1, b1, W2, b2):
    x = hidden_states.reshape(NTOK, DD)
    style8 = jnp.zeros((8, DD), jnp.float32).at[:BB].set(style_emb)
    pos0, pos1, w0b, w1b, te, usage = _router(x, style8, Wg)
    pos0 = pos0.reshape(NTOK)
    pos1 = pos1.reshape(NTOK)
    xs, ws = _dispatch_kernel()(x, pos0, pos1, w0b, w1b)
    ys = _grouped_mm(te.reshape(NTILES), xs, W1,
                     b1.reshape(EE, 1, II), W2, b2.reshape(EE, 1, DD), ws)
    out = _combine_kernel()(ys, pos0, pos1)
    return out.reshape(BB, LL, DD), usage.reshape(EE)
